# baseline (speedup 1.0000x reference)
import functools

import jax
import jax.numpy as jnp
from jax import lax
from jax.experimental import pallas as pl
from jax.experimental.pallas import tpu as pltpu
from jax.experimental.pallas import tpu_sc as plsc

_EMB = 256
_NEMB = 8192
_M_BLK = 1024


_RG = 8
_CG = 128


def _dist_argmin_body(x_ref, w_ref, colsq_ref, idx_ref):
    x = x_ref[...]
    rowsq = jnp.sum(x * x, axis=1, keepdims=True)
    dot2 = jnp.dot(x + x, w_ref[...])
    colsq = jnp.broadcast_to(colsq_ref[...], (_RG, _NEMB))
    lane = lax.broadcasted_iota(jnp.int32, (_RG, _CG), 1).astype(jnp.float32)
    n_cg = _NEMB // _CG
    for r in range(0, _M_BLK, _RG):
        rs = rowsq[r:r + _RG, :]
        m_v = jnp.full((_RG, _CG), jnp.inf, dtype=jnp.float32)
        g_v = jnp.zeros((_RG, _CG), dtype=jnp.float32)
        for g in range(n_cg):
            d = (rs - dot2[r:r + _RG, g * _CG:(g + 1) * _CG]
                 ) + colsq[:, g * _CG:(g + 1) * _CG]
            upd = d < m_v
            m_v = jnp.minimum(m_v, d)
            g_v = jnp.where(upd, float(g), g_v)
        m = jnp.min(m_v, axis=1, keepdims=True)
        cand = jnp.where(m_v == m, g_v * float(_CG) + lane, float(_NEMB))
        first = jnp.min(cand, axis=1, keepdims=True)
        idx_ref[r:r + _RG, :] = first.astype(jnp.int32)


def _argmin_indices(flat, w):
    n_rows = flat.shape[0]
    grid = (n_rows // _M_BLK,)
    return pl.pallas_call(
        _dist_argmin_body,
        grid=grid,
        in_specs=[
            pl.BlockSpec((_M_BLK, _EMB), lambda i: (i, 0)),
            pl.BlockSpec((_EMB, _NEMB), lambda i: (0, 0)),
            pl.BlockSpec((1, _NEMB), lambda i: (0, 0)),
        ],
        out_specs=pl.BlockSpec((_M_BLK, 1), lambda i: (i, 0)),
        out_shape=jax.ShapeDtypeStruct((n_rows, 1), jnp.int32),
        compiler_params=pltpu.CompilerParams(
            dimension_semantics=("parallel",),
        ),
    )(flat, w, jnp.sum(w ** 2, axis=0, keepdims=True))


def _make_sc_gather(B, D):
    info = plsc.get_sparse_core_info()
    NC, NS = info.num_cores, info.num_subcores
    NW = NC * NS
    b_per_w = B // NW
    n_chunks = b_per_w // 128
    mesh = plsc.VectorSubcoreMesh(core_axis_name="c", subcore_axis_name="s")

    @functools.partial(
        pl.kernel, mesh=mesh,
        out_type=jax.ShapeDtypeStruct((B, D), jnp.float32),
        scratch_types=[
            pltpu.VMEM((n_chunks, 128), jnp.int32),
            pltpu.VMEM((b_per_w, D), jnp.float32),
            pltpu.SemaphoreType.DMA,
            pltpu.SemaphoreType.DMA,
        ],
    )
    def gather_k(table_hbm, idx_hbm, out_hbm, idx_v, rows_v, sem, wsem):
        wid = lax.axis_index("s") * NC + lax.axis_index("c")
        base = wid * b_per_w
        for c in range(n_chunks):
            pltpu.sync_copy(idx_hbm.at[pl.ds(base + c * 128, 128)],
                            idx_v.at[c])
        gathers = [
            pltpu.async_copy(table_hbm.at[idx_v.at[c]],
                             rows_v.at[pl.ds(c * 128, 128)], sem)
            for c in range(n_chunks)
        ]
        writes = []
        for c in range(n_chunks):
            gathers[c].wait()
            writes.append(pltpu.async_copy(
                rows_v.at[pl.ds(c * 128, 128)],
                out_hbm.at[pl.ds(base + c * 128, 128)], wsem))
        for wcp in writes:
            wcp.wait()

    return gather_k


def kernel(x, w):
    flat = jnp.reshape(x, (-1, _EMB))
    idx = _argmin_indices(flat, w)
    wt = jnp.transpose(w)
    gather_k = _make_sc_gather(_NEMB, _EMB)
    quant = gather_k(wt, jnp.reshape(idx, (-1,)))
    return jnp.reshape(quant, x.shape)


# speedup vs baseline: 1.0920x; 1.0920x over previous
import functools

import jax
import jax.numpy as jnp
from jax import lax
from jax.experimental import pallas as pl
from jax.experimental.pallas import tpu as pltpu
from jax.experimental.pallas import tpu_sc as plsc

_EMB = 256
_NEMB = 8192
_M_BLK = 1024


_RG = 8
_CG = 128


def _dist_argmin_body(x_ref, w_ref, idx_ref, wt_ref, colsq_ref):
    @pl.when(pl.program_id(0) == 0)
    def _():
        w0 = w_ref[...]
        colsq_ref[...] = jnp.broadcast_to(
            jnp.sum(w0 * w0, axis=0, keepdims=True), (_RG, _NEMB))
        wt_ref[...] = jnp.transpose(w0)

    x = x_ref[...]
    rowsq = jnp.sum(x * x, axis=1, keepdims=True)
    dot2 = jnp.dot(x + x, w_ref[...])
    colsq = colsq_ref[...]
    lane = lax.broadcasted_iota(jnp.int32, (_RG, _CG), 1).astype(jnp.float32)
    n_cg = _NEMB // _CG
    for r in range(0, _M_BLK, _RG):
        rs = rowsq[r:r + _RG, :]
        m_v = jnp.full((_RG, _CG), jnp.inf, dtype=jnp.float32)
        g_v = jnp.zeros((_RG, _CG), dtype=jnp.float32)
        for g in range(n_cg):
            d = (rs - dot2[r:r + _RG, g * _CG:(g + 1) * _CG]
                 ) + colsq[:, g * _CG:(g + 1) * _CG]
            upd = d < m_v
            m_v = jnp.minimum(m_v, d)
            g_v = jnp.where(upd, float(g), g_v)
        m = jnp.min(m_v, axis=1, keepdims=True)
        cand = jnp.where(m_v == m, g_v * float(_CG) + lane, float(_NEMB))
        first = jnp.min(cand, axis=1, keepdims=True)
        idx_ref[r:r + _RG, :] = first.astype(jnp.int32)


def _argmin_indices(flat, w):
    n_rows = flat.shape[0]
    grid = (n_rows // _M_BLK,)
    return pl.pallas_call(
        _dist_argmin_body,
        grid=grid,
        in_specs=[
            pl.BlockSpec((_M_BLK, _EMB), lambda i: (i, 0)),
            pl.BlockSpec((_EMB, _NEMB), lambda i: (0, 0)),
        ],
        out_specs=[
            pl.BlockSpec((_M_BLK, 1), lambda i: (i, 0)),
            pl.BlockSpec((_NEMB, _EMB), lambda i: (0, 0)),
        ],
        out_shape=[
            jax.ShapeDtypeStruct((n_rows, 1), jnp.int32),
            jax.ShapeDtypeStruct((_NEMB, _EMB), jnp.float32),
        ],
        scratch_shapes=[pltpu.VMEM((_RG, _NEMB), jnp.float32)],
        compiler_params=pltpu.CompilerParams(
            dimension_semantics=("arbitrary",),
        ),
    )(flat, w)


def _make_sc_gather(B, D):
    info = plsc.get_sparse_core_info()
    NC, NS = info.num_cores, info.num_subcores
    NW = NC * NS
    b_per_w = B // NW
    n_chunks = b_per_w // 128
    mesh = plsc.VectorSubcoreMesh(core_axis_name="c", subcore_axis_name="s")

    @functools.partial(
        pl.kernel, mesh=mesh,
        out_type=jax.ShapeDtypeStruct((B, D), jnp.float32),
        scratch_types=[
            pltpu.VMEM((n_chunks, 128), jnp.int32),
            pltpu.VMEM((b_per_w, D), jnp.float32),
            pltpu.SemaphoreType.DMA,
            pltpu.SemaphoreType.DMA,
        ],
    )
    def gather_k(table_hbm, idx_hbm, out_hbm, idx_v, rows_v, sem, wsem):
        wid = lax.axis_index("s") * NC + lax.axis_index("c")
        base = wid * b_per_w
        for c in range(n_chunks):
            pltpu.sync_copy(idx_hbm.at[pl.ds(base + c * 128, 128)],
                            idx_v.at[c])
        gathers = [
            pltpu.async_copy(table_hbm.at[idx_v.at[c]],
                             rows_v.at[pl.ds(c * 128, 128)], sem)
            for c in range(n_chunks)
        ]
        writes = []
        for c in range(n_chunks):
            gathers[c].wait()
            writes.append(pltpu.async_copy(
                rows_v.at[pl.ds(c * 128, 128)],
                out_hbm.at[pl.ds(base + c * 128, 128)], wsem))
        for wcp in writes:
            wcp.wait()

    return gather_k


def kernel(x, w):
    flat = jnp.reshape(x, (-1, _EMB))
    idx, wt = _argmin_indices(flat, w)
    gather_k = _make_sc_gather(_NEMB, _EMB)
    quant = gather_k(wt, jnp.reshape(idx, (-1,)))
    return jnp.reshape(quant, x.shape)
